# baseline (speedup 1.0000x reference)
import functools

import jax
import jax.numpy as jnp
import numpy as np
from jax import lax
from jax.experimental import pallas as pl
from jax.experimental.pallas import tpu as pltpu
from jax.experimental.pallas import tpu_sc as plsc

N = 10000
E = 320000
D = 128

NUM_CORES = 2
NUM_SUBCORES = 16
NUM_WORKERS = NUM_CORES * NUM_SUBCORES
EPW = E // NUM_WORKERS
K = 128
NCH = E // K // NUM_WORKERS
NBIG = E // K - NUM_WORKERS * NCH
NTRI = (NCH + 1 + 2) // 3
RPT = 624
REMR = N - NUM_SUBCORES * RPT
LANES = 16
_SPLAT = [np.full((LANES,), l, np.int32) for l in range(LANES)]


EB = 2 * E // 10


def _linear_body(x_ref, w_ref, b_ref, ei_ref, o_ref, eo_ref):
    o_ref[...] = (
        lax.dot_general(x_ref[...], w_ref[...], (((1,), (1,)), ((), ())),
                        preferred_element_type=jnp.float32)
        + b_ref[...]
    )
    eo_ref[pl.ds(0, EB)] = ei_ref[0, :]
    eo_ref[pl.ds(EB, EB)] = ei_ref[1, :]


def _linear(x, w, b2, ei):
    bn = 2000
    return pl.pallas_call(
        _linear_body,
        grid=(N // bn,),
        in_specs=[
            pl.BlockSpec((bn, D), lambda i: (i, 0)),
            pl.BlockSpec((D, D), lambda i: (0, 0)),
            pl.BlockSpec((1, D), lambda i: (0, 0)),
            pl.BlockSpec((2, EB), lambda i: (0, i)),
        ],
        out_specs=[
            pl.BlockSpec((bn, D), lambda i: (i, 0)),
            pl.BlockSpec((2 * EB,), lambda i: (i,)),
        ],
        out_shape=[
            jax.ShapeDtypeStruct((N, D), jnp.float32),
            jax.ShapeDtypeStruct((2 * E,), jnp.int32),
        ],
    )(x, w, b2, ei)


def _combine_body(p_ref, o_ref):
    o_ref[...] = p_ref[0] + p_ref[1]


def _combine(partials):
    bn = 2000
    return pl.pallas_call(
        _combine_body,
        grid=(N // bn,),
        in_specs=[pl.BlockSpec((NUM_CORES, bn, D), lambda i: (0, i, 0))],
        out_specs=pl.BlockSpec((bn, D), lambda i: (i, 0)),
        out_shape=jax.ShapeDtypeStruct((N, D), jnp.float32),
    )(partials)


def _sc_body(h_hbm, ei_hbm, w_hbm, out_hbm,
             sbuf0, sbuf1, sbuf2, didx0, didx1, didx2, wbuf0, wbuf1, wbuf2,
             dsx0, dsx1, dsx2, rows0, rows1, rows2, acc,
             sem_e0, sem_e1, sem_e2, sem_g0, sem_g1, sem_g2,
             sem_s0, sem_s1, sem_s2):
    cid = lax.axis_index("c")
    sid = lax.axis_index("s")
    wid = sid * NUM_CORES + cid
    big = wid < NBIG
    cnt = jnp.where(big, NCH + 1, NCH)
    tb = jnp.where(big, wid * (NCH + 1) * K, NBIG * K + wid * NCH * K)

    sbuf = [sbuf0, sbuf1, sbuf2]
    didx = [didx0, didx1, didx2]
    dsx = [dsx0, dsx1, dsx2]
    wbuf = [wbuf0, wbuf1, wbuf2]
    rows = [rows0, rows1, rows2]
    sem_e = [sem_e0, sem_e1, sem_e2]
    sem_g = [sem_g0, sem_g1, sem_g2]
    sem_s = [sem_s0, sem_s1, sem_s2]

    def e_copies(c, b):
        off = tb + c * K
        blk = off // EB
        pos = blk * (2 * EB) + (off - blk * EB)
        return [
            pltpu.make_async_copy(ei_hbm.at[pl.ds(pos, K)], sbuf[b],
                                  sem_e[b]),
            pltpu.make_async_copy(ei_hbm.at[pl.ds(pos + EB, K)], didx[b],
                                  sem_e[b]),
            pltpu.make_async_copy(w_hbm.at[pl.ds(off, K)], wbuf[b],
                                  sem_e[b]),
        ]

    def g_copy(b):
        return pltpu.make_async_copy(h_hbm.at[sbuf[b]], rows[b], sem_g[b])

    def s_copy(b):
        return pltpu.make_async_copy(rows[b], acc.at[dsx[b]], sem_s[b])

    def snap_didx(b):
        for j in range(K // LANES):
            sl = pl.ds(j * LANES, LANES)
            dsx[b][sl] = didx[b][sl]

    def scale(rws, wref, ngrp):
        @plsc.parallel_loop(0, ngrp, step=1, unroll=4)
        def _grp(g):
            wv = wref[pl.ds(g * LANES, LANES)]
            for l in range(LANES):
                ws = wv[l]
                for j in range(D // LANES):
                    sl = pl.ds(j * LANES, LANES)
                    rws[g * LANES + l, sl] = rws[g * LANES + l, sl] * ws

    for cp in e_copies(0, 0):
        cp.start()
    for cp in e_copies(1, 1):
        cp.start()

    def zrow(r, carry):
        def zcol(j, c2):
            rows0[r, pl.ds(j * LANES, LANES)] = jnp.zeros((LANES,),
                                                          jnp.float32)
            return c2
        return lax.fori_loop(0, D // LANES, zcol, carry)
    lax.fori_loop(0, K, zrow, 0)

    rbase = sid * RPT
    zc = [pltpu.make_async_copy(rows0, acc.at[pl.ds(rbase + t * K, K)],
                                sem_g2) for t in range(RPT // K)]
    zc.append(pltpu.make_async_copy(
        rows0.at[pl.ds(0, RPT - (RPT // K) * K)],
        acc.at[pl.ds(rbase + (RPT // K) * K, RPT - (RPT // K) * K)],
        sem_g2))
    for cp in zc:
        cp.start()

    @pl.when(sid == NUM_SUBCORES - 1)
    def _zero_rem():
        pltpu.sync_copy(rows0.at[pl.ds(0, REMR)],
                        acc.at[pl.ds(NUM_SUBCORES * RPT, REMR)])

    for cp in e_copies(0, 0):
        cp.wait()
    for cp in zc:
        cp.wait()
    g_copy(0).start()

    plsc.subcore_barrier()

    def tri(t, carry):
        for b in range(3):
            c = 3 * t + b
            bn = (b + 1) % 3
            bnn = (b + 2) % 3

            @pl.when(c < cnt)
            def _chunk():
                @pl.when(c >= 2)
                def _wait_s():
                    s_copy(bn).wait()

                @pl.when(c + 1 < cnt)
                def _next_g():
                    for cp in e_copies(c + 1, bn):
                        cp.wait()
                    g_copy(bn).start()

                @pl.when(c + 2 < cnt)
                def _next_e():
                    for cp in e_copies(c + 2, bnn):
                        cp.start()

                g_copy(b).wait()
                snap_didx(b)
                scale(rows[b], wbuf[b], K // LANES)
                pltpu.async_copy(rows[b], acc.at[dsx[b]], sem_s[b],
                                 add=True)
        return carry

    lax.fori_loop(0, NTRI, tri, 0)

    @pl.when(big)
    def _drain_big():
        s_copy((NCH - 1) % 3).wait()
        s_copy(NCH % 3).wait()

    @pl.when(jnp.logical_not(big))
    def _drain_small():
        s_copy((NCH - 2) % 3).wait()
        s_copy((NCH - 1) % 3).wait()

    plsc.subcore_barrier()

    pltpu.sync_copy(acc.at[pl.ds(rbase, RPT)],
                    out_hbm.at[cid, pl.ds(rbase, RPT)])

    @pl.when(sid == NUM_SUBCORES - 1)
    def _write_rem():
        pltpu.sync_copy(acc.at[pl.ds(NUM_SUBCORES * RPT, REMR)],
                        out_hbm.at[cid, pl.ds(NUM_SUBCORES * RPT, REMR)])


_sc_scatter = functools.partial(
    pl.kernel,
    out_type=jax.ShapeDtypeStruct((NUM_CORES, N, D), jnp.float32),
    mesh=plsc.VectorSubcoreMesh(core_axis_name="c", subcore_axis_name="s"),
    scratch_types=[
        pltpu.VMEM((K,), jnp.int32),
        pltpu.VMEM((K,), jnp.int32),
        pltpu.VMEM((K,), jnp.int32),
        pltpu.VMEM((K,), jnp.int32),
        pltpu.VMEM((K,), jnp.int32),
        pltpu.VMEM((K,), jnp.int32),
        pltpu.VMEM((K,), jnp.float32),
        pltpu.VMEM((K,), jnp.float32),
        pltpu.VMEM((K,), jnp.float32),
        pltpu.VMEM((K,), jnp.int32),
        pltpu.VMEM((K,), jnp.int32),
        pltpu.VMEM((K,), jnp.int32),
        pltpu.VMEM((K, D), jnp.float32),
        pltpu.VMEM((K, D), jnp.float32),
        pltpu.VMEM((K, D), jnp.float32),
        pltpu.VMEM_SHARED((N, D), jnp.float32),
        pltpu.SemaphoreType.DMA,
        pltpu.SemaphoreType.DMA,
        pltpu.SemaphoreType.DMA,
        pltpu.SemaphoreType.DMA,
        pltpu.SemaphoreType.DMA,
        pltpu.SemaphoreType.DMA,
        pltpu.SemaphoreType.DMA,
        pltpu.SemaphoreType.DMA,
        pltpu.SemaphoreType.DMA,
    ],
)(_sc_body)


@jax.jit
def kernel(x, edge_index, edge_weight, W, b):
    h, ei_il = _linear(x, W, b.reshape(1, D), edge_index)
    partials = _sc_scatter(h, ei_il, edge_weight)
    return _combine(partials)


# speedup vs baseline: 1.0358x; 1.0358x over previous
import functools

import jax
import jax.numpy as jnp
import numpy as np
from jax import lax
from jax.experimental import pallas as pl
from jax.experimental.pallas import tpu as pltpu
from jax.experimental.pallas import tpu_sc as plsc

N = 10000
E = 320000
D = 128

NUM_CORES = 2
NUM_SUBCORES = 16
NUM_WORKERS = NUM_CORES * NUM_SUBCORES
EPW = E // NUM_WORKERS
K = 128
NCH = E // K // NUM_WORKERS
NBIG = E // K - NUM_WORKERS * NCH
NTRI = (NCH + 1 + 2) // 3
RPT = 624
REMR = N - NUM_SUBCORES * RPT
LANES = 16
_SPLAT = [np.full((LANES,), l, np.int32) for l in range(LANES)]


EB = 2 * E // 10


def _linear_body(x_ref, w_ref, b_ref, ei_ref, o_ref, eo_ref):
    o_ref[...] = (
        lax.dot_general(x_ref[...], w_ref[...], (((1,), (1,)), ((), ())),
                        preferred_element_type=jnp.float32)
        + b_ref[...]
    )
    eo_ref[pl.ds(0, EB)] = ei_ref[0, :]
    eo_ref[pl.ds(EB, EB)] = ei_ref[1, :]


def _linear(x, w, b2, ei):
    bn = 2000
    return pl.pallas_call(
        _linear_body,
        grid=(N // bn,),
        in_specs=[
            pl.BlockSpec((bn, D), lambda i: (i, 0)),
            pl.BlockSpec((D, D), lambda i: (0, 0)),
            pl.BlockSpec((1, D), lambda i: (0, 0)),
            pl.BlockSpec((2, EB), lambda i: (0, i)),
        ],
        out_specs=[
            pl.BlockSpec((bn, D), lambda i: (i, 0)),
            pl.BlockSpec((2 * EB,), lambda i: (i,)),
        ],
        out_shape=[
            jax.ShapeDtypeStruct((N, D), jnp.float32),
            jax.ShapeDtypeStruct((2 * E,), jnp.int32),
        ],
    )(x, w, b2, ei)


def _combine_body(p_ref, o_ref):
    o_ref[...] = p_ref[0] + p_ref[1]


def _combine(partials):
    bn = 2000
    return pl.pallas_call(
        _combine_body,
        grid=(N // bn,),
        in_specs=[pl.BlockSpec((NUM_CORES, bn, D), lambda i: (0, i, 0))],
        out_specs=pl.BlockSpec((bn, D), lambda i: (i, 0)),
        out_shape=jax.ShapeDtypeStruct((N, D), jnp.float32),
    )(partials)


def _sc_body(h_hbm, ei_hbm, w_hbm, out_hbm,
             sbuf0, sbuf1, sbuf2, didx0, didx1, didx2, wbuf0, wbuf1, wbuf2,
             dsx0, dsx1, dsx2, rows0, rows1, rows2, acc,
             sem_e0, sem_e1, sem_e2, sem_g0, sem_g1, sem_g2,
             sem_s0, sem_s1, sem_s2):
    cid = lax.axis_index("c")
    sid = lax.axis_index("s")
    wid = sid * NUM_CORES + cid
    big = wid < NBIG
    cnt = jnp.where(big, NCH + 1, NCH)
    tb = jnp.where(big, wid * (NCH + 1) * K, NBIG * K + wid * NCH * K)

    sbuf = [sbuf0, sbuf1, sbuf2]
    didx = [didx0, didx1, didx2]
    dsx = [dsx0, dsx1, dsx2]
    wbuf = [wbuf0, wbuf1, wbuf2]
    rows = [rows0, rows1, rows2]
    sem_e = [sem_e0, sem_e1, sem_e2]
    sem_g = [sem_g0, sem_g1, sem_g2]
    sem_s = [sem_s0, sem_s1, sem_s2]

    def e_copies(c, b):
        off = tb + c * K
        blk = off // EB
        pos = blk * (2 * EB) + (off - blk * EB)
        return [
            pltpu.make_async_copy(ei_hbm.at[pl.ds(pos, K)], sbuf[b],
                                  sem_e[b]),
            pltpu.make_async_copy(ei_hbm.at[pl.ds(pos + EB, K)], didx[b],
                                  sem_e[b]),
            pltpu.make_async_copy(w_hbm.at[pl.ds(off, K)], wbuf[b],
                                  sem_e[b]),
        ]

    def g_copy(b):
        return pltpu.make_async_copy(h_hbm.at[sbuf[b]], rows[b], sem_g[b])

    def s_copy(b):
        return pltpu.make_async_copy(rows[b], acc.at[dsx[b]], sem_s[b])

    def snap_didx(b):
        for j in range(K // LANES):
            sl = pl.ds(j * LANES, LANES)
            dsx[b][sl] = didx[b][sl]

    def scale(rws, wref, ngrp):
        @plsc.parallel_loop(0, ngrp, step=1, unroll=1)
        def _grp(g):
            wv = wref[pl.ds(g * LANES, LANES)]
            for l in range(LANES):
                ws = wv[l]
                for j in range(D // LANES):
                    sl = pl.ds(j * LANES, LANES)
                    rws[g * LANES + l, sl] = rws[g * LANES + l, sl] * ws

    for cp in e_copies(0, 0):
        cp.start()
    for cp in e_copies(1, 1):
        cp.start()

    def zrow(r, carry):
        def zcol(j, c2):
            rows0[r, pl.ds(j * LANES, LANES)] = jnp.zeros((LANES,),
                                                          jnp.float32)
            return c2
        return lax.fori_loop(0, D // LANES, zcol, carry)
    lax.fori_loop(0, K, zrow, 0)

    rbase = sid * RPT
    zc = [pltpu.make_async_copy(rows0, acc.at[pl.ds(rbase + t * K, K)],
                                sem_g2) for t in range(RPT // K)]
    zc.append(pltpu.make_async_copy(
        rows0.at[pl.ds(0, RPT - (RPT // K) * K)],
        acc.at[pl.ds(rbase + (RPT // K) * K, RPT - (RPT // K) * K)],
        sem_g2))
    for cp in zc:
        cp.start()

    @pl.when(sid == NUM_SUBCORES - 1)
    def _zero_rem():
        pltpu.sync_copy(rows0.at[pl.ds(0, REMR)],
                        acc.at[pl.ds(NUM_SUBCORES * RPT, REMR)])

    for cp in e_copies(0, 0):
        cp.wait()
    for cp in zc:
        cp.wait()
    g_copy(0).start()

    plsc.subcore_barrier()

    def tri(t, carry):
        for b in range(3):
            c = 3 * t + b
            bn = (b + 1) % 3
            bnn = (b + 2) % 3

            @pl.when(c < cnt)
            def _chunk():
                @pl.when(c >= 2)
                def _wait_s():
                    s_copy(bn).wait()

                @pl.when(c + 1 < cnt)
                def _next_g():
                    for cp in e_copies(c + 1, bn):
                        cp.wait()
                    g_copy(bn).start()

                @pl.when(c + 2 < cnt)
                def _next_e():
                    for cp in e_copies(c + 2, bnn):
                        cp.start()

                g_copy(b).wait()
                snap_didx(b)
                scale(rows[b], wbuf[b], K // LANES)
                pltpu.async_copy(rows[b], acc.at[dsx[b]], sem_s[b],
                                 add=True)
        return carry

    lax.fori_loop(0, NTRI, tri, 0)

    @pl.when(big)
    def _drain_big():
        s_copy((NCH - 1) % 3).wait()
        s_copy(NCH % 3).wait()

    @pl.when(jnp.logical_not(big))
    def _drain_small():
        s_copy((NCH - 2) % 3).wait()
        s_copy((NCH - 1) % 3).wait()

    plsc.subcore_barrier()

    pltpu.sync_copy(acc.at[pl.ds(rbase, RPT)],
                    out_hbm.at[cid, pl.ds(rbase, RPT)])

    @pl.when(sid == NUM_SUBCORES - 1)
    def _write_rem():
        pltpu.sync_copy(acc.at[pl.ds(NUM_SUBCORES * RPT, REMR)],
                        out_hbm.at[cid, pl.ds(NUM_SUBCORES * RPT, REMR)])


_sc_scatter = functools.partial(
    pl.kernel,
    out_type=jax.ShapeDtypeStruct((NUM_CORES, N, D), jnp.float32),
    mesh=plsc.VectorSubcoreMesh(core_axis_name="c", subcore_axis_name="s"),
    scratch_types=[
        pltpu.VMEM((K,), jnp.int32),
        pltpu.VMEM((K,), jnp.int32),
        pltpu.VMEM((K,), jnp.int32),
        pltpu.VMEM((K,), jnp.int32),
        pltpu.VMEM((K,), jnp.int32),
        pltpu.VMEM((K,), jnp.int32),
        pltpu.VMEM((K,), jnp.float32),
        pltpu.VMEM((K,), jnp.float32),
        pltpu.VMEM((K,), jnp.float32),
        pltpu.VMEM((K,), jnp.int32),
        pltpu.VMEM((K,), jnp.int32),
        pltpu.VMEM((K,), jnp.int32),
        pltpu.VMEM((K, D), jnp.float32),
        pltpu.VMEM((K, D), jnp.float32),
        pltpu.VMEM((K, D), jnp.float32),
        pltpu.VMEM_SHARED((N, D), jnp.float32),
        pltpu.SemaphoreType.DMA,
        pltpu.SemaphoreType.DMA,
        pltpu.SemaphoreType.DMA,
        pltpu.SemaphoreType.DMA,
        pltpu.SemaphoreType.DMA,
        pltpu.SemaphoreType.DMA,
        pltpu.SemaphoreType.DMA,
        pltpu.SemaphoreType.DMA,
        pltpu.SemaphoreType.DMA,
    ],
)(_sc_body)


@jax.jit
def kernel(x, edge_index, edge_weight, W, b):
    h, ei_il = _linear(x, W, b.reshape(1, D), edge_index)
    partials = _sc_scatter(h, ei_il, edge_weight)
    return _combine(partials)


# speedup vs baseline: 1.0933x; 1.0555x over previous
import functools

import jax
import jax.numpy as jnp
import numpy as np
from jax import lax
from jax.experimental import pallas as pl
from jax.experimental.pallas import tpu as pltpu
from jax.experimental.pallas import tpu_sc as plsc

N = 10000
E = 320000
D = 128

NUM_CORES = 2
NUM_SUBCORES = 16
NUM_WORKERS = NUM_CORES * NUM_SUBCORES
EPW = E // NUM_WORKERS
K = 128
NCH = E // K // NUM_WORKERS
NBIG = E // K - NUM_WORKERS * NCH
NTRI = (NCH + 1 + 2) // 3
RPT = 624
REMR = N - NUM_SUBCORES * RPT
LANES = 16
_SPLAT = [np.full((LANES,), l, np.int32) for l in range(LANES)]


EB = 2 * E // 10


def _linear_body(x_ref, w_ref, b_ref, ei_ref, o_ref, eo_ref):
    o_ref[...] = (
        lax.dot_general(x_ref[...], w_ref[...], (((1,), (1,)), ((), ())),
                        preferred_element_type=jnp.float32)
        + b_ref[...]
    )
    eo_ref[pl.ds(0, EB)] = ei_ref[0, :]
    eo_ref[pl.ds(EB, EB)] = ei_ref[1, :]


def _linear(x, w, b2, ei):
    bn = 2000
    return pl.pallas_call(
        _linear_body,
        grid=(N // bn,),
        in_specs=[
            pl.BlockSpec((bn, D), lambda i: (i, 0)),
            pl.BlockSpec((D, D), lambda i: (0, 0)),
            pl.BlockSpec((1, D), lambda i: (0, 0)),
            pl.BlockSpec((2, EB), lambda i: (0, i)),
        ],
        out_specs=[
            pl.BlockSpec((bn, D), lambda i: (i, 0)),
            pl.BlockSpec((2 * EB,), lambda i: (i,)),
        ],
        out_shape=[
            jax.ShapeDtypeStruct((N, D), jnp.float32),
            jax.ShapeDtypeStruct((2 * E,), jnp.int32),
        ],
    )(x, w, b2, ei)


def _combine_body(p_ref, o_ref):
    o_ref[...] = p_ref[0] + p_ref[1]


def _combine(partials):
    bn = 2000
    return pl.pallas_call(
        _combine_body,
        grid=(N // bn,),
        in_specs=[pl.BlockSpec((NUM_CORES, bn, D), lambda i: (0, i, 0))],
        out_specs=pl.BlockSpec((bn, D), lambda i: (i, 0)),
        out_shape=jax.ShapeDtypeStruct((N, D), jnp.float32),
    )(partials)


def _sc_body(h_hbm, ei_hbm, w_hbm, out_hbm,
             sbuf0, sbuf1, sbuf2, didx0, didx1, didx2, wbuf0, wbuf1, wbuf2,
             dsx0, dsx1, dsx2, rows0, rows1, rows2, acc,
             sem_e0, sem_e1, sem_e2, sem_g0, sem_g1, sem_g2,
             sem_s0, sem_s1, sem_s2):
    cid = lax.axis_index("c")
    sid = lax.axis_index("s")
    wid = sid * NUM_CORES + cid
    big = wid < NBIG
    cnt = jnp.where(big, NCH + 1, NCH)
    tb = jnp.where(big, wid * (NCH + 1) * K, NBIG * K + wid * NCH * K)

    sbuf = [sbuf0, sbuf1, sbuf2]
    didx = [didx0, didx1, didx2]
    dsx = [dsx0, dsx1, dsx2]
    wbuf = [wbuf0, wbuf1, wbuf2]
    rows = [rows0, rows1, rows2]
    sem_e = [sem_e0, sem_e1, sem_e2]
    sem_g = [sem_g0, sem_g1, sem_g2]
    sem_s = [sem_s0, sem_s1, sem_s2]

    def e_copies(c, b):
        off = tb + c * K
        blk = off // EB
        pos = blk * (2 * EB) + (off - blk * EB)
        return [
            pltpu.make_async_copy(ei_hbm.at[pl.ds(pos, K)], sbuf[b],
                                  sem_e[b]),
            pltpu.make_async_copy(ei_hbm.at[pl.ds(pos + EB, K)], didx[b],
                                  sem_e[b]),
            pltpu.make_async_copy(w_hbm.at[pl.ds(off, K)], wbuf[b],
                                  sem_e[b]),
        ]

    def g_copy(b):
        return pltpu.make_async_copy(h_hbm.at[sbuf[b]], rows[b], sem_g[b])

    def s_copy(b):
        return pltpu.make_async_copy(rows[b], acc.at[dsx[b]], sem_s[b])

    def snap_didx(b):
        for j in range(K // LANES):
            sl = pl.ds(j * LANES, LANES)
            dsx[b][sl] = didx[b][sl]

    def scale(rws, wref, ngrp):
        @plsc.parallel_loop(0, ngrp, step=1, unroll=2)
        def _grp(g):
            wv = wref[pl.ds(g * LANES, LANES)]
            for l in range(LANES):
                ws = wv[l]
                for j in range(D // LANES):
                    sl = pl.ds(j * LANES, LANES)
                    rws[g * LANES + l, sl] = rws[g * LANES + l, sl] * ws

    for cp in e_copies(0, 0):
        cp.start()
    for cp in e_copies(1, 1):
        cp.start()

    def zrow(r, carry):
        def zcol(j, c2):
            rows0[r, pl.ds(j * LANES, LANES)] = jnp.zeros((LANES,),
                                                          jnp.float32)
            return c2
        return lax.fori_loop(0, D // LANES, zcol, carry)
    lax.fori_loop(0, K, zrow, 0)

    rbase = sid * RPT
    zc = [pltpu.make_async_copy(rows0, acc.at[pl.ds(rbase + t * K, K)],
                                sem_g2) for t in range(RPT // K)]
    zc.append(pltpu.make_async_copy(
        rows0.at[pl.ds(0, RPT - (RPT // K) * K)],
        acc.at[pl.ds(rbase + (RPT // K) * K, RPT - (RPT // K) * K)],
        sem_g2))
    for cp in zc:
        cp.start()

    @pl.when(sid == NUM_SUBCORES - 1)
    def _zero_rem():
        pltpu.sync_copy(rows0.at[pl.ds(0, REMR)],
                        acc.at[pl.ds(NUM_SUBCORES * RPT, REMR)])

    for cp in e_copies(0, 0):
        cp.wait()
    for cp in zc:
        cp.wait()
    g_copy(0).start()

    plsc.subcore_barrier()

    def tri(t, carry):
        for b in range(3):
            c = 3 * t + b
            bn = (b + 1) % 3
            bnn = (b + 2) % 3

            @pl.when(c < cnt)
            def _chunk():
                @pl.when(c >= 2)
                def _wait_s():
                    s_copy(bn).wait()

                @pl.when(c + 1 < cnt)
                def _next_g():
                    for cp in e_copies(c + 1, bn):
                        cp.wait()
                    g_copy(bn).start()

                @pl.when(c + 2 < cnt)
                def _next_e():
                    for cp in e_copies(c + 2, bnn):
                        cp.start()

                g_copy(b).wait()
                snap_didx(b)
                scale(rows[b], wbuf[b], K // LANES)
                pltpu.async_copy(rows[b], acc.at[dsx[b]], sem_s[b],
                                 add=True)
        return carry

    lax.fori_loop(0, NTRI, tri, 0)

    @pl.when(big)
    def _drain_big():
        s_copy((NCH - 1) % 3).wait()
        s_copy(NCH % 3).wait()

    @pl.when(jnp.logical_not(big))
    def _drain_small():
        s_copy((NCH - 2) % 3).wait()
        s_copy((NCH - 1) % 3).wait()

    plsc.subcore_barrier()

    pltpu.sync_copy(acc.at[pl.ds(rbase, RPT)],
                    out_hbm.at[cid, pl.ds(rbase, RPT)])

    @pl.when(sid == NUM_SUBCORES - 1)
    def _write_rem():
        pltpu.sync_copy(acc.at[pl.ds(NUM_SUBCORES * RPT, REMR)],
                        out_hbm.at[cid, pl.ds(NUM_SUBCORES * RPT, REMR)])


_sc_scatter = functools.partial(
    pl.kernel,
    out_type=jax.ShapeDtypeStruct((NUM_CORES, N, D), jnp.float32),
    mesh=plsc.VectorSubcoreMesh(core_axis_name="c", subcore_axis_name="s"),
    scratch_types=[
        pltpu.VMEM((K,), jnp.int32),
        pltpu.VMEM((K,), jnp.int32),
        pltpu.VMEM((K,), jnp.int32),
        pltpu.VMEM((K,), jnp.int32),
        pltpu.VMEM((K,), jnp.int32),
        pltpu.VMEM((K,), jnp.int32),
        pltpu.VMEM((K,), jnp.float32),
        pltpu.VMEM((K,), jnp.float32),
        pltpu.VMEM((K,), jnp.float32),
        pltpu.VMEM((K,), jnp.int32),
        pltpu.VMEM((K,), jnp.int32),
        pltpu.VMEM((K,), jnp.int32),
        pltpu.VMEM((K, D), jnp.float32),
        pltpu.VMEM((K, D), jnp.float32),
        pltpu.VMEM((K, D), jnp.float32),
        pltpu.VMEM_SHARED((N, D), jnp.float32),
        pltpu.SemaphoreType.DMA,
        pltpu.SemaphoreType.DMA,
        pltpu.SemaphoreType.DMA,
        pltpu.SemaphoreType.DMA,
        pltpu.SemaphoreType.DMA,
        pltpu.SemaphoreType.DMA,
        pltpu.SemaphoreType.DMA,
        pltpu.SemaphoreType.DMA,
        pltpu.SemaphoreType.DMA,
    ],
)(_sc_body)


@jax.jit
def kernel(x, edge_index, edge_weight, W, b):
    h, ei_il = _linear(x, W, b.reshape(1, D), edge_index)
    partials = _sc_scatter(h, ei_il, edge_weight)
    return _combine(partials)
